# Initial kernel scaffold; baseline (speedup 1.0000x reference)
import jax
import jax.numpy as jnp
from jax.experimental import pallas as pl


def kernel(in_feat, edge_index, W1, b1, W2, b2, Wc1, bc1, Wc2, bc2, W3, b3, W4, b4):
    raise NotImplementedError("write your pallas kernel here")


# speedup vs baseline: 4.1205x; 4.1205x over previous
import functools

import jax
import jax.numpy as jnp
from jax import lax
from jax.experimental import pallas as pl
from jax.experimental.pallas import tpu as pltpu
from jax.experimental.pallas import tpu_sc as plsc

N = 10000
E = 320000
D = 128
C = 2

NSC = 2
NSUB = 16
NT = NSC * NSUB
EPT = E // NT
CH = 80
NCH = EPT // CH
NP = 10240
RPT = NP // NSUB
DEGW = 128

_mesh = plsc.VectorSubcoreMesh(core_axis_name="c", subcore_axis_name="s")


@functools.partial(
    pl.kernel,
    out_type=jax.ShapeDtypeStruct((NSC, NP, D), jnp.float32),
    mesh=_mesh,
    scratch_types=[
        pltpu.VMEM((CH,), jnp.int32),
        pltpu.VMEM((CH,), jnp.int32),
        pltpu.VMEM((CH, D), jnp.float32),
        pltpu.VMEM_SHARED((NP, D), jnp.float32),
        pltpu.SemaphoreType.DMA,
    ],
)
def _sc_scatter_rows(x_hbm, src_hbm, dst_hbm, zeros_hbm, out_hbm,
                     idx_s, idx_d, rows, acc, sem):
    c = lax.axis_index("c")
    s = lax.axis_index("s")
    r0 = s * RPT
    pltpu.sync_copy(zeros_hbm.at[pl.ds(r0, RPT)], acc.at[pl.ds(r0, RPT)])
    plsc.subcore_barrier()
    base = (c * NSUB + s) * EPT

    def body(k, carry):
        off = base + k * CH
        pltpu.sync_copy(src_hbm.at[pl.ds(off, CH)], idx_s)
        pltpu.sync_copy(dst_hbm.at[pl.ds(off, CH)], idx_d)
        pltpu.async_copy(x_hbm.at[idx_s], rows, sem).wait()
        pltpu.sync_copy(rows, acc.at[idx_d], add=True)
        return carry

    lax.fori_loop(0, NCH, body, 0)
    plsc.subcore_barrier()
    pltpu.sync_copy(acc.at[pl.ds(r0, RPT)], out_hbm.at[c, pl.ds(r0, RPT)])


@functools.partial(
    pl.kernel,
    out_type=jax.ShapeDtypeStruct((NSC, NP, DEGW), jnp.float32),
    mesh=_mesh,
    scratch_types=[
        pltpu.VMEM((CH,), jnp.int32),
        pltpu.VMEM((CH, DEGW), jnp.float32),
        pltpu.VMEM_SHARED((NP, DEGW), jnp.float32),
    ],
)
def _sc_degree(dst_hbm, ones_hbm, zeros_hbm, out_hbm, idx_d, ones_v, acc):
    c = lax.axis_index("c")
    s = lax.axis_index("s")
    r0 = s * RPT
    pltpu.sync_copy(zeros_hbm.at[pl.ds(r0, RPT)], acc.at[pl.ds(r0, RPT)])
    pltpu.sync_copy(ones_hbm, ones_v)
    plsc.subcore_barrier()
    base = (c * NSUB + s) * EPT

    def body(k, carry):
        off = base + k * CH
        pltpu.sync_copy(dst_hbm.at[pl.ds(off, CH)], idx_d)
        pltpu.sync_copy(ones_v, acc.at[idx_d], add=True)
        return carry

    lax.fori_loop(0, NCH, body, 0)
    plsc.subcore_barrier()
    pltpu.sync_copy(acc.at[pl.ds(r0, RPT)], out_hbm.at[c, pl.ds(r0, RPT)])


R = 400
GRID = N // R

_row = pl.BlockSpec((R, D), lambda i: (i, 0))
_row2 = pl.BlockSpec((NSC, R, D), lambda i: (0, i, 0))
_col1 = pl.BlockSpec((R, 1), lambda i: (i, 0))
_full = lambda *shape: pl.BlockSpec(shape, lambda i: (0,) * len(shape))


def _dot(a, b):
    return jnp.dot(a, b, preferred_element_type=jnp.float32)


def _tc_mlp_body(x_ref, w1_ref, b1_ref, w2_ref, b2_ref, h_ref):
    h1 = jnp.maximum(_dot(x_ref[...], w1_ref[...]) + b1_ref[...], 0.0)
    h_ref[...] = jnp.maximum(_dot(h1, w2_ref[...]) + b2_ref[...], 0.0)


_tc_mlp = pl.pallas_call(
    _tc_mlp_body,
    grid=(GRID,),
    in_specs=[_row, _full(D, D), _full(1, D), _full(D, D), _full(1, D)],
    out_specs=_row,
    out_shape=jax.ShapeDtypeStruct((N, D), jnp.float32),
)


def _tc_prescale_body(h_ref, degp_ref, y0_ref, dinv_ref):
    deg = degp_ref[0, :, 0:1] + degp_ref[1, :, 0:1]
    dinv = lax.rsqrt(jnp.maximum(deg, 1.0))
    dinv_ref[...] = dinv
    y0_ref[...] = h_ref[...] * dinv


_tc_prescale = pl.pallas_call(
    _tc_prescale_body,
    grid=(GRID,),
    in_specs=[_row, pl.BlockSpec((NSC, R, DEGW), lambda i: (0, i, 0))],
    out_specs=[_row, _col1],
    out_shape=[
        jax.ShapeDtypeStruct((N, D), jnp.float32),
        jax.ShapeDtypeStruct((N, 1), jnp.float32),
    ],
)


def _tc_x1_body(g_ref, dinv_ref, x1_ref, y1_ref):
    dinv = dinv_ref[...]
    x1 = (g_ref[0] + g_ref[1]) * (-dinv)
    x1_ref[...] = x1
    y1_ref[...] = x1 * dinv


_tc_x1 = pl.pallas_call(
    _tc_x1_body,
    grid=(GRID,),
    in_specs=[_row2, _col1],
    out_specs=[_row, _row],
    out_shape=[
        jax.ShapeDtypeStruct((N, D), jnp.float32),
        jax.ShapeDtypeStruct((N, D), jnp.float32),
    ],
)


def _cheb_combine(h, x1, g_ref, dinv, wc_ref, bc_ref):
    x2 = (g_ref[0] + g_ref[1]) * (-2.0 * dinv) - h
    acc = _dot(h, wc_ref[0:D])
    acc = acc + _dot(x1, wc_ref[D:2 * D])
    acc = acc + _dot(x2, wc_ref[2 * D:3 * D])
    return jnp.maximum(acc + bc_ref[...], 0.0)


def _tc_cheb_out_body(h_ref, x1_ref, g_ref, dinv_ref, wc_ref, bc_ref,
                      hn_ref, y0n_ref):
    dinv = dinv_ref[...]
    hn = _cheb_combine(h_ref[...], x1_ref[...], g_ref, dinv, wc_ref, bc_ref)
    hn_ref[...] = hn
    y0n_ref[...] = hn * dinv


_tc_cheb_out = pl.pallas_call(
    _tc_cheb_out_body,
    grid=(GRID,),
    in_specs=[_row, _row, _row2, _col1, _full(3 * D, D), _full(1, D)],
    out_specs=[_row, _row],
    out_shape=[
        jax.ShapeDtypeStruct((N, D), jnp.float32),
        jax.ShapeDtypeStruct((N, D), jnp.float32),
    ],
)


def _tc_head_body(h_ref, x1_ref, g_ref, dinv_ref, wc_ref, bc_ref,
                  w3_ref, b3_ref, w4_ref, b4_ref, out_ref):
    hn = _cheb_combine(h_ref[...], x1_ref[...], g_ref, dinv_ref[...],
                       wc_ref, bc_ref)
    t = jnp.maximum(_dot(hn, w3_ref[...]) + b3_ref[...], 0.0)
    out_ref[...] = _dot(t, w4_ref[...]) + b4_ref[...]


_tc_head = pl.pallas_call(
    _tc_head_body,
    grid=(GRID,),
    in_specs=[_row, _row, _row2, _col1, _full(3 * D, D), _full(1, D),
              _full(D, D), _full(1, D), _full(D, C), _full(1, C)],
    out_specs=pl.BlockSpec((R, C), lambda i: (i, 0)),
    out_shape=jax.ShapeDtypeStruct((N, C), jnp.float32),
)


def kernel(in_feat, edge_index, W1, b1, W2, b2, Wc1, bc1, Wc2, bc2,
           W3, b3, W4, b4):
    src = edge_index[0]
    dst = edge_index[1]
    zeros_d = jnp.zeros((NP, D), jnp.float32)
    zeros_g = jnp.zeros((NP, DEGW), jnp.float32)
    ones_g = jnp.ones((CH, DEGW), jnp.float32)
    b1r, b2r = b1.reshape(1, D), b2.reshape(1, D)
    bc1r, bc2r = bc1.reshape(1, D), bc2.reshape(1, D)
    b3r, b4r = b3.reshape(1, D), b4.reshape(1, C)

    degp = _sc_degree(dst, ones_g, zeros_g)
    h = _tc_mlp(in_feat, W1, b1r, W2, b2r)
    y0, dinv = _tc_prescale(h, degp)

    g0 = _sc_scatter_rows(y0, src, dst, zeros_d)
    x1, y1 = _tc_x1(g0, dinv)
    g1 = _sc_scatter_rows(y1, src, dst, zeros_d)
    h2, y0b = _tc_cheb_out(h, x1, g1, dinv, Wc1, bc1r)

    g2 = _sc_scatter_rows(y0b, src, dst, zeros_d)
    x1b, y1b = _tc_x1(g2, dinv)
    g3 = _sc_scatter_rows(y1b, src, dst, zeros_d)
    return _tc_head(h2, x1b, g3, dinv, Wc2, bc2r, W3, b3r, W4, b4r)
